# baseline (speedup 1.0000x reference)
import functools

import jax
import jax.numpy as jnp
from jax import lax
from jax.experimental import pallas as pl
from jax.experimental.pallas import tpu as pltpu

_K = 10
_W_TIME = 0.3
_DECAY = 0.9
_SIGMA = 300.0
_LOG_INV_DECAY = float(jnp.log(jnp.float32(1.0) / jnp.float32(_DECAY)))


def _msg_weights(t, t0, mask):
  mf = mask.astype(jnp.float32)
  ii = lax.broadcasted_iota(jnp.int32, (_K, _K), 0)
  jj = lax.broadcasted_iota(jnp.int32, (_K, _K), 1)
  tri = (ii <= jj).astype(jnp.float32)
  cum = jnp.dot(mf, tri, preferred_element_type=jnp.float32) - 1.0
  geo = jnp.where(mask, jnp.exp(cum * _LOG_INV_DECAY), 0.0)
  kern = jnp.exp(-((t - t0) ** 2) / (2.0 * _SIGMA * _SIGMA))
  return geo * kern, jnp.sum(geo)


def _gather_msg_body(ids_sm, times_sm, v_ref, t_ref, valid_ref,
                     msg_ref, fmsg_ref):
  b = pl.program_id(0)
  t0 = times_sm[b].astype(jnp.float32)
  t = t_ref[0].astype(jnp.float32)
  valid = valid_ref[0] != 0
  v = v_ref[0]

  for past, out_ref in ((True, msg_ref), (False, fmsg_ref)):
    cond = (t < t0) if past else (t > t0)
    w, den = _msg_weights(t, t0, valid & cond)
    num = jnp.dot(w, v, preferred_element_type=jnp.float32)
    msg = jnp.where(den > 0.0, num / jnp.maximum(den, 1e-12), 0.0)
    out_ref[0] = msg * _W_TIME


def _dense_body(a_ref, aa_ref, tgt_ref, msg_ref, fmsg_ref, qa_ref, loss_ref,
                *, bb, nsteps, denom):
  i = pl.program_id(0)

  a = a_ref[...]
  tgt = tgt_ref[...]
  rows = []
  for s in range(bb):
    aa_s = aa_ref[s]
    m2 = msg_ref[s:s + 1, :]
    f2 = fmsg_ref[s:s + 1, :]
    rowc = jnp.dot(m2, aa_s, preferred_element_type=jnp.float32)
    colc = lax.dot_general(f2, aa_s, (((1,), (1,)), ((), ())),
                           preferred_element_type=jnp.float32)
    rows.append(rowc + colc)
  contrib = jnp.concatenate(rows, axis=0)

  qa = jax.nn.sigmoid(a + contrib)
  qa_ref[...] = qa

  def bce_sum(p, t):
    p = jnp.clip(p, 1e-7, 1.0 - 1e-7)
    return -jnp.sum(t * jnp.log(p) + (1.0 - t) * jnp.log1p(-p),
                    keepdims=True)

  part = bce_sum(qa, tgt) + bce_sum(jax.nn.sigmoid(a), tgt)

  @pl.when(i == 0)
  def _init():
    loss_ref[...] = jnp.zeros_like(loss_ref)

  loss_ref[...] += part * denom


def _scatter_body(ids_sm, times_sm, a_ref, vorig_ref,
                  mv_in, mt_in, mvalid_in, mv_out, mt_out, mvalid_out):
  b = pl.program_id(0)
  slot = jnp.sum(vorig_ref[0]) % _K
  siga = jax.nn.sigmoid(a_ref[0])
  iota_k1 = lax.broadcasted_iota(jnp.int32, (_K, 1), 0)
  iota_1k = lax.broadcasted_iota(jnp.int32, (1, _K), 1)
  mv_out[0] = jnp.where(iota_k1 == slot, siga, mv_in[0])
  mt_out[0] = jnp.where(iota_1k == slot, times_sm[b], mt_in[0])
  mvalid_out[0] = jnp.where(iota_1k == slot, 1, mvalid_in[0])


def kernel(a, aa, target, ids, times, mem_values, mem_times, mem_valid):
  B, C = a.shape
  M = mem_values.shape[0]
  ids = ids.astype(jnp.int32)
  times = times.astype(jnp.int32)
  mt3 = mem_times.astype(jnp.int32).reshape(M, 1, _K)
  mvalid3 = mem_valid.astype(jnp.int32).reshape(M, 1, _K)

  msg3, fmsg3 = pl.pallas_call(
      _gather_msg_body,
      grid_spec=pltpu.PrefetchScalarGridSpec(
          num_scalar_prefetch=2,
          grid=(B,),
          in_specs=[
              pl.BlockSpec((1, _K, C), lambda b, ids, tm: (ids[b], 0, 0)),
              pl.BlockSpec((1, 1, _K), lambda b, ids, tm: (ids[b], 0, 0)),
              pl.BlockSpec((1, 1, _K), lambda b, ids, tm: (ids[b], 0, 0)),
          ],
          out_specs=[
              pl.BlockSpec((1, 1, C), lambda b, ids, tm: (b, 0, 0)),
              pl.BlockSpec((1, 1, C), lambda b, ids, tm: (b, 0, 0)),
          ],
      ),
      out_shape=[
          jax.ShapeDtypeStruct((B, 1, C), jnp.float32),
          jax.ShapeDtypeStruct((B, 1, C), jnp.float32),
      ],
  )(ids, times, mem_values, mt3, mvalid3)
  msg = msg3.reshape(B, C)
  fmsg = fmsg3.reshape(B, C)

  BB = 8
  nsteps = B // BB
  qa, loss11 = pl.pallas_call(
      functools.partial(_dense_body, bb=BB, nsteps=nsteps,
                        denom=1.0 / (3.0 * B * C)),
      grid=(nsteps,),
      in_specs=[
          pl.BlockSpec((BB, C), lambda i: (i, 0)),
          pl.BlockSpec((BB, C, C), lambda i: (i, 0, 0)),
          pl.BlockSpec((BB, C), lambda i: (i, 0)),
          pl.BlockSpec((BB, C), lambda i: (i, 0)),
          pl.BlockSpec((BB, C), lambda i: (i, 0)),
      ],
      out_specs=[
          pl.BlockSpec((BB, C), lambda i: (i, 0)),
          pl.BlockSpec((1, 1), lambda i: (0, 0)),
      ],
      out_shape=[
          jax.ShapeDtypeStruct((B, C), jnp.float32),
          jax.ShapeDtypeStruct((1, 1), jnp.float32),
      ],
  )(a, aa, target, msg, fmsg)
  loss = loss11.reshape(())

  a3 = a.reshape(B, 1, C)
  new_mv, new_mt3, new_mvalid3 = pl.pallas_call(
      _scatter_body,
      grid_spec=pltpu.PrefetchScalarGridSpec(
          num_scalar_prefetch=2,
          grid=(B,),
          in_specs=[
              pl.BlockSpec((1, 1, C), lambda b, ids, tm: (b, 0, 0)),
              pl.BlockSpec((1, 1, _K), lambda b, ids, tm: (ids[b], 0, 0)),
              pl.BlockSpec((1, _K, C), lambda b, ids, tm: (ids[b], 0, 0)),
              pl.BlockSpec((1, 1, _K), lambda b, ids, tm: (ids[b], 0, 0)),
              pl.BlockSpec((1, 1, _K), lambda b, ids, tm: (ids[b], 0, 0)),
          ],
          out_specs=[
              pl.BlockSpec((1, _K, C), lambda b, ids, tm: (ids[b], 0, 0)),
              pl.BlockSpec((1, 1, _K), lambda b, ids, tm: (ids[b], 0, 0)),
              pl.BlockSpec((1, 1, _K), lambda b, ids, tm: (ids[b], 0, 0)),
          ],
      ),
      out_shape=[
          jax.ShapeDtypeStruct((M, _K, C), jnp.float32),
          jax.ShapeDtypeStruct((M, 1, _K), jnp.int32),
          jax.ShapeDtypeStruct((M, 1, _K), jnp.int32),
      ],
      input_output_aliases={4: 0, 5: 1, 6: 2},
  )(ids, times, a3, mvalid3, mem_values, mt3, mvalid3)

  new_mem_times = new_mt3.reshape(M, _K).astype(mem_times.dtype)
  new_mem_valid = new_mvalid3.reshape(M, _K) != 0
  return (qa, loss, new_mv, new_mem_times, new_mem_valid)
